# baseline (speedup 1.0000x reference)
import functools

import jax
import jax.numpy as jnp
from jax import lax
from jax.experimental import pallas as pl
from jax.experimental.pallas import tpu as pltpu
from jax.experimental.pallas import tpu_sc as plsc

_NC, _NS, _L = 2, 16, 16


def _sc_body(k_hbm, v_hbm, idx_hbm, kb_hbm, vb_hbm, out_hbm, idx_v,
             *, m, nb, rows):
    wid = lax.axis_index("s") * _NC + lax.axis_index("c")
    base = wid * rows
    pltpu.sync_copy(kb_hbm.at[pl.ds(base, rows)], out_hbm.at[pl.ds(base, rows)])
    pltpu.sync_copy(vb_hbm.at[pl.ds(base, rows)],
                    out_hbm.at[pl.ds(m + base, rows)])
    pltpu.sync_copy(idx_hbm, idx_v)
    for c in range(nb // _L):
        chunk = idx_v[pl.ds(c * _L, _L)]
        for j in range(_L):
            b = c * _L + j
            t = chunk[j]

            @pl.when((t >= base) & (t < base + rows))
            def _():
                pltpu.sync_copy(k_hbm.at[pl.ds(b, 1)], out_hbm.at[pl.ds(t, 1)])
                pltpu.sync_copy(v_hbm.at[pl.ds(b, 1)],
                                out_hbm.at[pl.ds(m + t, 1)])


def kernel(k, v, mem_index, key_buffer, value_buffer):
    m, h, d = key_buffer.shape
    nb = k.shape[0]
    hd = h * d
    rows = m // (_NC * _NS)
    body = functools.partial(_sc_body, m=m, nb=nb, rows=rows)
    mesh = plsc.VectorSubcoreMesh(core_axis_name="c", subcore_axis_name="s")
    out = pl.kernel(
        body,
        out_type=jax.ShapeDtypeStruct((2 * m, hd), key_buffer.dtype),
        mesh=mesh,
        scratch_types=[pltpu.VMEM((nb,), jnp.int32)],
    )(k.reshape(nb, hd), v.reshape(nb, hd), mem_index.astype(jnp.int32),
      key_buffer.reshape(m, hd), value_buffer.reshape(m, hd))
    return out.reshape(2, m, h, d)


# speedup vs baseline: 14.2566x; 14.2566x over previous
import functools

import jax
import jax.numpy as jnp
from jax import lax
from jax.experimental import pallas as pl
from jax.experimental.pallas import tpu as pltpu
from jax.experimental.pallas import tpu_sc as plsc

_NC, _NS, _L = 2, 16, 16


def _copy_body(kb_ref, vb_ref, out_ref):
    out_ref[0] = kb_ref[...]
    out_ref[1] = vb_ref[...]


def _scatter_body(k_hbm, v_hbm, idx_hbm, out_hbm, idx_v, *, m, nb, rows):
    wid = lax.axis_index("s") * _NC + lax.axis_index("c")
    base = wid * rows
    pltpu.sync_copy(idx_hbm, idx_v)
    for c in range(nb // _L):
        chunk = idx_v[pl.ds(c * _L, _L)]
        for j in range(_L):
            b = c * _L + j
            t = chunk[j]

            @pl.when((t >= base) & (t < base + rows))
            def _():
                pltpu.sync_copy(k_hbm.at[pl.ds(b, 1)], out_hbm.at[pl.ds(t, 1)])
                pltpu.sync_copy(v_hbm.at[pl.ds(b, 1)],
                                out_hbm.at[pl.ds(m + t, 1)])


def kernel(k, v, mem_index, key_buffer, value_buffer):
    m, h, d = key_buffer.shape
    nb = k.shape[0]
    hd = h * d
    bm = min(1024, m)
    out = pl.pallas_call(
        _copy_body,
        grid=(m // bm,),
        in_specs=[
            pl.BlockSpec((bm, h, d), lambda i: (i, 0, 0)),
            pl.BlockSpec((bm, h, d), lambda i: (i, 0, 0)),
        ],
        out_specs=pl.BlockSpec((2, bm, h, d), lambda i: (0, i, 0, 0)),
        out_shape=jax.ShapeDtypeStruct((2, m, h, d), key_buffer.dtype),
    )(key_buffer, value_buffer)

    out_ref = jax.new_ref(out.reshape(2 * m, hd))
    rows = m // (_NC * _NS)
    body = functools.partial(_scatter_body, m=m, nb=nb, rows=rows)
    mesh = plsc.VectorSubcoreMesh(core_axis_name="c", subcore_axis_name="s")
    pl.kernel(
        body,
        out_type=(),
        mesh=mesh,
        scratch_types=[pltpu.VMEM((nb,), jnp.int32)],
    )(k.reshape(nb, hd), v.reshape(nb, hd), mem_index.astype(jnp.int32),
      out_ref)
    return out_ref[...].reshape(2, m, h, d)
